# baseline (speedup 1.0000x reference)
import functools

import jax
import jax.numpy as jnp
from jax import lax
from jax.experimental import pallas as pl
from jax.experimental.pallas import tpu as pltpu
from jax.experimental.pallas import tpu_sc as plsc

D_MODEL = 64
SCALE = 8.0

NC = 2
NS = 16
NW = NC * NS
CHUNK = 128


@functools.lru_cache(maxsize=None)
def _build(nchunks: int):
    mesh = plsc.VectorSubcoreMesh(core_axis_name="c", subcore_axis_name="s")

    @functools.partial(
        pl.kernel,
        mesh=mesh,
        out_type=jax.ShapeDtypeStruct((NW, nchunks, CHUNK, D_MODEL), jnp.float32),
        scratch_types=[
            pltpu.VMEM((nchunks, CHUNK), jnp.int32),
            pltpu.VMEM((CHUNK, D_MODEL), jnp.float32),
            pltpu.SemaphoreType.DMA,
        ],
        compiler_params=pltpu.CompilerParams(use_tc_tiling_on_sc=False),
    )
    def emb(x_hbm, table_hbm, out_hbm, idx_v, rows_v, sem):
        wid = lax.axis_index("s") * NC + lax.axis_index("c")
        pltpu.sync_copy(x_hbm.at[wid], idx_v)

        def chunk_body(c, carry):
            pltpu.async_copy(table_hbm.at[idx_v.at[c]], rows_v, sem).wait()

            def row_body(i, carry2):
                for j in range(D_MODEL // 16):
                    sl = pl.ds(j * 16, 16)
                    rows_v[i, sl] = rows_v[i, sl] * SCALE
                return carry2

            lax.fori_loop(0, CHUNK, row_body, 0, unroll=4)
            pltpu.sync_copy(rows_v, out_hbm.at[wid, c])
            return carry

        lax.fori_loop(0, nchunks, chunk_body, 0)

    return emb


def kernel(x, table):
    s0, s1 = x.shape
    total = s0 * s1
    assert total % (NW * CHUNK) == 0
    nchunks = total // (NW * CHUNK)
    xr = x.astype(jnp.int32).reshape(NW, nchunks, CHUNK)
    out = _build(nchunks)(xr, table)
    return out.reshape(s0, s1, D_MODEL)


# speedup vs baseline: 1.0541x; 1.0541x over previous
import functools

import jax
import jax.numpy as jnp
from jax import lax
from jax.experimental import pallas as pl
from jax.experimental.pallas import tpu as pltpu
from jax.experimental.pallas import tpu_sc as plsc

D_MODEL = 64
SCALE = 8.0

NC = 2
NS = 16
NW = NC * NS
CHUNK = 128
DEPTH = 4


@functools.lru_cache(maxsize=None)
def _build(nchunks: int):
    assert nchunks % DEPTH == 0 and nchunks // DEPTH >= 3
    mesh = plsc.VectorSubcoreMesh(core_axis_name="c", subcore_axis_name="s")

    @functools.partial(
        pl.kernel,
        mesh=mesh,
        out_type=jax.ShapeDtypeStruct((NW, nchunks, CHUNK, D_MODEL), jnp.float32),
        scratch_types=[
            pltpu.VMEM((nchunks, CHUNK), jnp.int32),
            pltpu.VMEM((DEPTH, CHUNK, D_MODEL), jnp.float32),
            pltpu.VMEM((DEPTH, CHUNK, D_MODEL), jnp.float32),
            pltpu.SemaphoreType.DMA((DEPTH,)),
            pltpu.SemaphoreType.DMA((DEPTH,)),
        ],
        compiler_params=pltpu.CompilerParams(use_tc_tiling_on_sc=False),
    )
    def emb(x_hbm, table_hbm, out_hbm, idx_v, in_v, out_v, gsem, psem):
        wid = lax.axis_index("s") * NC + lax.axis_index("c")
        pltpu.sync_copy(x_hbm.at[wid], idx_v)

        def gather_start(c, b):
            pltpu.async_copy(table_hbm.at[idx_v.at[c]], in_v.at[b], gsem.at[b])

        def gather_wait(c, b):
            pltpu.make_async_copy(
                table_hbm.at[idx_v.at[c]], in_v.at[b], gsem.at[b]
            ).wait()

        def put_start(c, b):
            pltpu.async_copy(out_v.at[b], out_hbm.at[wid, c], psem.at[b])

        def put_wait(c, b):
            pltpu.make_async_copy(
                out_v.at[b], out_hbm.at[wid, c], psem.at[b]
            ).wait()

        def scale(b):
            def row(i, carry):
                for j in range(D_MODEL // 16):
                    sl = pl.ds(j * 16, 16)
                    out_v[b, i, sl] = in_v[b, i, sl] * SCALE
                return carry

            lax.fori_loop(0, CHUNK, row, 0, unroll=2)

        for b in range(DEPTH):
            gather_start(b, b)
        for b in range(DEPTH):
            gather_wait(b, b)
            scale(b)
            put_start(b, b)
            gather_start(b + DEPTH, b)

        def steady(g, carry):
            for b in range(DEPTH):
                c = DEPTH * g + b
                gather_wait(c, b)
                put_wait(c - DEPTH, b)
                scale(b)
                put_start(c, b)
                gather_start(c + DEPTH, b)
            return carry

        lax.fori_loop(1, nchunks // DEPTH - 1, steady, 0)

        tail = nchunks - DEPTH
        for b in range(DEPTH):
            c = tail + b
            gather_wait(c, b)
            put_wait(c - DEPTH, b)
            scale(b)
            put_start(c, b)
        for b in range(DEPTH):
            put_wait(tail + b, b)

    return emb


def kernel(x, table):
    s0, s1 = x.shape
    total = s0 * s1
    assert total % (NW * CHUNK) == 0
    nchunks = total // (NW * CHUNK)
    xr = x.astype(jnp.int32).reshape(NW, nchunks, CHUNK)
    out = _build(nchunks)(xr, table)
    return out.reshape(s0, s1, D_MODEL)


# speedup vs baseline: 1.1650x; 1.1052x over previous
import functools

import jax
import jax.numpy as jnp
from jax import lax
from jax.experimental import pallas as pl
from jax.experimental.pallas import tpu as pltpu
from jax.experimental.pallas import tpu_sc as plsc

D_MODEL = 64
SCALE = 8.0

NC = 2
NS = 16
NW = NC * NS
CHUNK = 128
DEPTH = 4


@functools.lru_cache(maxsize=None)
def _build(nchunks: int):
    assert nchunks % DEPTH == 0 and nchunks // DEPTH >= 3
    mesh = plsc.VectorSubcoreMesh(core_axis_name="c", subcore_axis_name="s")

    @functools.partial(
        pl.kernel,
        mesh=mesh,
        out_type=jax.ShapeDtypeStruct((NW, nchunks, CHUNK, D_MODEL), jnp.float32),
        scratch_types=[
            pltpu.VMEM((nchunks, CHUNK), jnp.int32),
            pltpu.VMEM((DEPTH, CHUNK, D_MODEL), jnp.float32),
            pltpu.VMEM((DEPTH, CHUNK, D_MODEL), jnp.float32),
            pltpu.SemaphoreType.DMA((DEPTH,)),
            pltpu.SemaphoreType.DMA((DEPTH,)),
        ],
        compiler_params=pltpu.CompilerParams(use_tc_tiling_on_sc=False),
    )
    def emb(x_hbm, table_hbm, out_hbm, idx_v, in_v, out_v, gsem, psem):
        wid = lax.axis_index("s") * NC + lax.axis_index("c")
        pltpu.sync_copy(x_hbm.at[wid], idx_v)

        def gather_start(c, b):
            pltpu.async_copy(table_hbm.at[idx_v.at[c]], in_v.at[b], gsem.at[b])

        def gather_wait(c, b):
            pltpu.make_async_copy(
                table_hbm.at[idx_v.at[c]], in_v.at[b], gsem.at[b]
            ).wait()

        def put_start(c, b):
            pltpu.async_copy(out_v.at[b], out_hbm.at[wid, c], psem.at[b])

        def put_wait(c, b):
            pltpu.make_async_copy(
                out_v.at[b], out_hbm.at[wid, c], psem.at[b]
            ).wait()

        def scale(b):
            @plsc.parallel_loop(0, CHUNK, 1, unroll=4)
            def row(i):
                for j in range(D_MODEL // 16):
                    sl = pl.ds(j * 16, 16)
                    out_v[b, i, sl] = in_v[b, i, sl] * SCALE

        for b in range(DEPTH):
            gather_start(b, b)
        for b in range(DEPTH):
            gather_wait(b, b)
            scale(b)
            put_start(b, b)
            gather_start(b + DEPTH, b)

        def steady(g, carry):
            for b in range(DEPTH):
                c = DEPTH * g + b
                gather_wait(c, b)
                put_wait(c - DEPTH, b)
                scale(b)
                put_start(c, b)
                gather_start(c + DEPTH, b)
            return carry

        lax.fori_loop(1, nchunks // DEPTH - 1, steady, 0)

        tail = nchunks - DEPTH
        for b in range(DEPTH):
            c = tail + b
            gather_wait(c, b)
            put_wait(c - DEPTH, b)
            scale(b)
            put_start(c, b)
        for b in range(DEPTH):
            put_wait(tail + b, b)

    return emb


def kernel(x, table):
    s0, s1 = x.shape
    total = s0 * s1
    assert total % (NW * CHUNK) == 0
    nchunks = total // (NW * CHUNK)
    xr = x.astype(jnp.int32).reshape(NW, nchunks, CHUNK)
    out = _build(nchunks)(xr, table)
    return out.reshape(s0, s1, D_MODEL)
